# Initial kernel scaffold; baseline (speedup 1.0000x reference)
import jax
import jax.numpy as jnp
from jax.experimental import pallas as pl


def kernel(all_node_emb, supernode_edge_index, supernode_idx, graph_batch, W, b):
    raise NotImplementedError("write your pallas kernel here")


# speedup vs baseline: 2.7696x; 2.7696x over previous
import functools

import jax
import jax.numpy as jnp
from jax import lax
from jax.experimental import pallas as pl
from jax.experimental.pallas import tpu as pltpu
from jax.experimental.pallas import tpu_sc as plsc

NEG_INF = float("-inf")


def _sc_segment(all_node_emb, graph_batch, B):
    N, D = all_node_emb.shape
    NS = 16
    SUB = 400
    NSUB = N // SUB
    BLOCKS = SUB // 16
    DV = D // 16
    GB = B // NS

    mesh = plsc.VectorSubcoreMesh(
        core_axis_name="c", subcore_axis_name="s", num_cores=1)

    out_type = (
        jax.ShapeDtypeStruct((B, D), jnp.float32),
        jax.ShapeDtypeStruct((B, D), jnp.float32),
        jax.ShapeDtypeStruct((B, D), jnp.float32),
    )

    scratch_types = [
        pltpu.VMEM((SUB, D), jnp.float32),
        pltpu.VMEM((SUB,), jnp.int32),
        pltpu.VMEM((16,), jnp.int32),
        pltpu.VMEM((D,), jnp.float32),
        pltpu.VMEM((16,), jnp.int32),
        pltpu.VMEM((GB, D), jnp.float32),
        pltpu.VMEM((GB,), jnp.int32),
        pltpu.VMEM((8 * B,), jnp.int32),
        pltpu.VMEM((B,), jnp.int32),
        pltpu.VMEM((B,), jnp.int32),
        pltpu.VMEM((B,), jnp.int32),
        pltpu.VMEM((16 * NS,), jnp.int32),
        pltpu.VMEM((2 * NS, D), jnp.float32),
        pltpu.SMEM((1,), jnp.int32),
        pltpu.SMEM((1,), jnp.int32),
        pltpu.SMEM((1,), jnp.int32),
        pltpu.SemaphoreType.DMA,
        pltpu.VMEM_SHARED((B, D), jnp.float32),
        pltpu.VMEM_SHARED((8 * B,), jnp.int32),
        pltpu.VMEM_SHARED((16 * NS,), jnp.int32),
        pltpu.VMEM_SHARED((2 * NS, D), jnp.float32),
        pltpu.VMEM_SHARED((B,), jnp.int32),
        pltpu.VMEM_SHARED((B,), jnp.int32),
    ]

    @functools.partial(
        pl.kernel, out_type=out_type, mesh=mesh, scratch_types=scratch_types,
        compiler_params=pltpu.CompilerParams(needs_layout_passes=False))
    def sck(emb_hbm, batch_hbm, hout, tout, mout,
            emb_v, ids_v, prev_v, acc_v, stage_v, row_v, idx_v,
            hb_v, hc_v, hg_v, tg_v, eseg_v, emax_v,
            cur_s, fs_s, pv_s, sem,
            maxout_sh, head_sh, eseg_sh, emax_sh, hg_sh, tg_sh):
        t = lax.axis_index("s")

        minf16 = jnp.full((16,), NEG_INF, jnp.float32)

        def fill_row(j, _):
            for d in range(DV):
                row_v[j, pl.ds(16 * d, 16)] = minf16
            return 0
        lax.fori_loop(0, GB, fill_row, 0)
        pltpu.sync_copy(row_v, maxout_sh.at[pl.ds(GB * t, GB)])

        nslot = 8 * B // NS
        nsplat = jnp.full((16,), N, jnp.int32)
        def fill_h(j, _):
            hb_v[pl.ds(16 * j, 16)] = nsplat
            return 0
        lax.fori_loop(0, nslot // 16, fill_h, 0)
        pltpu.sync_copy(hb_v.at[pl.ds(0, nslot)],
                        head_sh.at[pl.ds(nslot * t, nslot)])

        plsc.subcore_barrier()

        s0 = (t * NSUB) // NS
        s1 = ((t + 1) * NSUB) // NS
        r0 = s0 * SUB

        @pl.when(t > 0)
        def _():
            pltpu.sync_copy(batch_hbm.at[pl.ds(r0 - 16, 16)], prev_v)
        pvv = prev_v[pl.ds(0, 16)]
        pv_s[0] = jnp.where(t > 0, pvv[15], -1)
        cur_s[0] = -1

        def reset_acc():
            for d in range(DV):
                acc_v[pl.ds(16 * d, 16)] = minf16

        reset_acc()

        def flush():
            cur = cur_s[0]
            @pl.when(cur == fs_s[0])
            def _():
                pltpu.sync_copy(acc_v, emax_sh.at[2 * t])
            @pl.when(cur != fs_s[0])
            def _():
                pltpu.sync_copy(acc_v, maxout_sh.at[cur])

        def headwrite(sid, g):
            stage_v[pl.ds(0, 16)] = jnp.full((16,), 1, jnp.int32) * g
            pltpu.sync_copy(stage_v.at[pl.ds(0, 8)],
                            head_sh.at[pl.ds(sid * 8, 8)])

        def boundary(sid, g):
            cur = cur_s[0]
            @pl.when(cur >= 0)
            def _():
                flush()
            prev_row = jnp.where(cur >= 0, cur, pv_s[0])
            @pl.when(sid != prev_row)
            def _():
                headwrite(sid, g)
            cur_s[0] = sid

        def accum_block(k):
            for d in range(DV):
                a = acc_v[pl.ds(16 * d, 16)]
                for r in range(16):
                    a = jnp.maximum(a, emb_v[16 * k + r, pl.ds(16 * d, 16)])
                acc_v[pl.ds(16 * d, 16)] = a

        def subchunk(si, _):
            row0 = (s0 + si) * SUB
            pltpu.sync_copy(emb_hbm.at[pl.ds(row0, SUB)], emb_v)
            pltpu.sync_copy(batch_hbm.at[pl.ds(row0, SUB)], ids_v)

            @pl.when(si == 0)
            def _():
                fs_s[0] = ids_v[pl.ds(0, 16)][0]

            def block(k, _):
                iv = ids_v[pl.ds(16 * k, 16)]
                mn = iv[0]
                mx = iv[15]
                g0 = row0 + 16 * k
                cur = cur_s[0]

                @pl.when(jnp.logical_and(mn == mx, mn == cur))
                def _():
                    accum_block(k)

                @pl.when(jnp.logical_and(mn == mx, mn != cur))
                def _():
                    boundary(mn, g0)
                    reset_acc()
                    accum_block(k)

                @pl.when(mn != mx)
                def _():
                    for r in range(16):
                        sid = iv[r]
                        @pl.when(sid != cur_s[0])
                        def _(sid=sid, r=r):
                            boundary(sid, g0 + r)
                            reset_acc()
                        for d in range(DV):
                            acc_v[pl.ds(16 * d, 16)] = jnp.maximum(
                                acc_v[pl.ds(16 * d, 16)],
                                emb_v[16 * k + r, pl.ds(16 * d, 16)])
                return 0

            lax.fori_loop(0, BLOCKS, block, 0)
            return 0

        lax.fori_loop(0, s1 - s0, subchunk, 0)

        stage_v[pl.ds(0, 16)] = jnp.full((16,), 1, jnp.int32) * fs_s[0]
        pltpu.sync_copy(stage_v.at[pl.ds(0, 8)],
                        eseg_sh.at[pl.ds(16 * t, 8)])
        stage_v[pl.ds(0, 16)] = jnp.full((16,), 1, jnp.int32) * cur_s[0]
        pltpu.sync_copy(stage_v.at[pl.ds(0, 8)],
                        eseg_sh.at[pl.ds(16 * t + 8, 8)])

        @pl.when(cur_s[0] == fs_s[0])
        def _():
            pltpu.sync_copy(acc_v, emax_sh.at[2 * t])
            pltpu.sync_copy(acc_v, emax_sh.at[2 * t + 1])

        @pl.when(cur_s[0] != fs_s[0])
        def _():
            pltpu.sync_copy(acc_v, emax_sh.at[2 * t + 1])

        plsc.subcore_barrier()

        @pl.when(t == 0)
        def _():
            pltpu.sync_copy(eseg_sh, eseg_v)
            pltpu.sync_copy(emax_sh, emax_v)

            iota16 = lax.iota(jnp.int32, 16)
            es0 = plsc.load_gather(eseg_v, [iota16 * 8])
            es1 = plsc.load_gather(eseg_v, [(iota16 + 16) * 8])

            cur_s[0] = es0[0]
            for d in range(DV):
                acc_v[pl.ds(16 * d, 16)] = emax_v[0, pl.ds(16 * d, 16)]

            for e in range(1, 2 * NS):
                s = es0[e] if e < 16 else es1[e - 16]
                @pl.when(s == cur_s[0])
                def _(s=s, e=e):
                    for d in range(DV):
                        acc_v[pl.ds(16 * d, 16)] = jnp.maximum(
                            acc_v[pl.ds(16 * d, 16)],
                            emax_v[e, pl.ds(16 * d, 16)])
                @pl.when(s != cur_s[0])
                def _(s=s, e=e):
                    pltpu.sync_copy(acc_v, maxout_sh.at[cur_s[0]])
                    cur_s[0] = s
                    for d in range(DV):
                        acc_v[pl.ds(16 * d, 16)] = emax_v[e, pl.ds(16 * d, 16)]
            pltpu.sync_copy(acc_v, maxout_sh.at[cur_s[0]])

            pltpu.sync_copy(head_sh, hb_v)
            iota16 = lax.iota(jnp.int32, 16)

            def bf(i, carry):
                ci = B // 16 - 1 - i
                idx = (ci * 16 + iota16) * 8
                v = plsc.load_gather(hb_v, [idx])
                r = lax.rev(-v, dimensions=(0,))
                c = jnp.maximum(plsc.cummax(r), carry)
                hc_v[pl.ds(ci * 16, 16)] = -lax.rev(c, dimensions=(0,))
                return c[15]
            lax.fori_loop(0, B // 16, bf, jnp.int32(-N))

            def gi(i, _):
                v = hc_v[pl.ds(16 * i, 16)]
                hg_v[pl.ds(16 * i, 16)] = jnp.minimum(v, N - 1)
                tg_v[pl.ds(16 * i, 16)] = jnp.minimum(v + 1, N - 1)
                return 0
            lax.fori_loop(0, B // 16, gi, 0)
            pltpu.sync_copy(hg_v, hg_sh)
            pltpu.sync_copy(tg_v, tg_sh)

        plsc.subcore_barrier()

        pltpu.sync_copy(hg_sh.at[pl.ds(GB * t, GB)], idx_v)
        pltpu.async_copy(emb_hbm.at[idx_v], row_v, sem).wait()
        pltpu.sync_copy(row_v, hout.at[pl.ds(GB * t, GB)])

        pltpu.sync_copy(tg_sh.at[pl.ds(GB * t, GB)], idx_v)
        pltpu.async_copy(emb_hbm.at[idx_v], row_v, sem).wait()
        pltpu.sync_copy(row_v, tout.at[pl.ds(GB * t, GB)])

        pltpu.sync_copy(maxout_sh.at[pl.ds(GB * t, GB)],
                        mout.at[pl.ds(GB * t, GB)])

    return sck(all_node_emb, graph_batch)


def _tc_proj(hrows, trows, mrows, W, b2):
    B, D = hrows.shape

    def body(h_ref, t_ref, m_ref, w_ref, b_ref, o_ref):
        w = w_ref[...]
        dn = (((1,), (1,)), ((), ()))
        acc = lax.dot_general(h_ref[...], w[:, 0:D], dn,
                              preferred_element_type=jnp.float32)
        acc += lax.dot_general(t_ref[...], w[:, D:2 * D], dn,
                               preferred_element_type=jnp.float32)
        acc += lax.dot_general(m_ref[...], w[:, 2 * D:3 * D], dn,
                               preferred_element_type=jnp.float32)
        o_ref[...] = acc + b_ref[...]

    return pl.pallas_call(
        body,
        out_shape=jax.ShapeDtypeStruct((B, D), jnp.float32),
    )(hrows, trows, mrows, W, b2)


def kernel(all_node_emb, supernode_edge_index, supernode_idx, graph_batch, W, b):
    B = supernode_idx.shape[0]
    D = all_node_emb.shape[1]
    hrows, trows, mrows = _sc_segment(all_node_emb, graph_batch, B)
    return _tc_proj(hrows, trows, mrows, W, b.reshape(1, D))


# speedup vs baseline: 4.0633x; 1.4671x over previous
import functools

import jax
import jax.numpy as jnp
from jax import lax
from jax.experimental import pallas as pl
from jax.experimental.pallas import tpu as pltpu
from jax.experimental.pallas import tpu_sc as plsc

NEG_INF = float("-inf")


def _sc_segment(all_node_emb, graph_batch, B):
    N, D = all_node_emb.shape
    NS = 16
    SUB = 160
    NSUB = N // SUB
    BLOCKS = SUB // 16
    DV = D // 16
    GB = B // NS

    mesh = plsc.VectorSubcoreMesh(
        core_axis_name="c", subcore_axis_name="s", num_cores=1)

    out_type = (
        jax.ShapeDtypeStruct((B, D), jnp.float32),
        jax.ShapeDtypeStruct((B, D), jnp.float32),
        jax.ShapeDtypeStruct((B, D), jnp.float32),
    )

    scratch_types = [
        pltpu.VMEM((SUB, D), jnp.float32),
        pltpu.VMEM((SUB, D), jnp.float32),
        pltpu.VMEM((SUB,), jnp.int32),
        pltpu.VMEM((SUB,), jnp.int32),
        pltpu.SemaphoreType.DMA,
        pltpu.SemaphoreType.DMA,
        pltpu.SemaphoreType.DMA,
        pltpu.SemaphoreType.DMA,
        pltpu.VMEM((16,), jnp.int32),
        pltpu.VMEM((D,), jnp.float32),
        pltpu.VMEM((16,), jnp.int32),
        pltpu.VMEM((GB, D), jnp.float32),
        pltpu.VMEM((GB,), jnp.int32),
        pltpu.VMEM((8 * B,), jnp.int32),
        pltpu.VMEM((B,), jnp.int32),
        pltpu.VMEM((B,), jnp.int32),
        pltpu.VMEM((B,), jnp.int32),
        pltpu.VMEM((16 * NS,), jnp.int32),
        pltpu.VMEM((2 * NS, D), jnp.float32),
        pltpu.SMEM((1,), jnp.int32),
        pltpu.SMEM((1,), jnp.int32),
        pltpu.SMEM((1,), jnp.int32),
        pltpu.SemaphoreType.DMA,
        pltpu.VMEM_SHARED((B, D), jnp.float32),
        pltpu.VMEM_SHARED((8 * B,), jnp.int32),
        pltpu.VMEM_SHARED((16 * NS,), jnp.int32),
        pltpu.VMEM_SHARED((2 * NS, D), jnp.float32),
        pltpu.VMEM_SHARED((B,), jnp.int32),
        pltpu.VMEM_SHARED((B,), jnp.int32),
    ]

    @functools.partial(
        pl.kernel, out_type=out_type, mesh=mesh, scratch_types=scratch_types,
        compiler_params=pltpu.CompilerParams(needs_layout_passes=False))
    def sck(emb_hbm, batch_hbm, hout, tout, mout,
            emb0_v, emb1_v, ids0_v, ids1_v, seme0, seme1, semi0, semi1,
            prev_v, acc_v, stage_v, row_v, idx_v,
            hb_v, hc_v, hg_v, tg_v, eseg_v, emax_v,
            cur_s, fs_s, pv_s, sem,
            maxout_sh, head_sh, eseg_sh, emax_sh, hg_sh, tg_sh):
        t = lax.axis_index("s")

        minf16 = jnp.full((16,), NEG_INF, jnp.float32)
        bufs = ((emb0_v, ids0_v, seme0, semi0), (emb1_v, ids1_v, seme1, semi1))

        s0 = (t * NSUB) // NS
        s1 = ((t + 1) * NSUB) // NS
        nsub = s1 - s0

        def dma_pair(si, eb, ib, se, sj):
            row0 = (s0 + si) * SUB
            return (pltpu.make_async_copy(emb_hbm.at[pl.ds(row0, SUB)], eb, se),
                    pltpu.make_async_copy(batch_hbm.at[pl.ds(row0, SUB)], ib, sj))

        for b in range(2):
            for cp in dma_pair(b, *bufs[b]):
                cp.start()

        def fill_row(j, _):
            for d in range(DV):
                row_v[j, pl.ds(16 * d, 16)] = minf16
            return 0
        lax.fori_loop(0, GB, fill_row, 0)
        pltpu.sync_copy(row_v, maxout_sh.at[pl.ds(GB * t, GB)])

        nslot = 8 * B // NS
        nsplat = jnp.full((16,), N, jnp.int32)
        def fill_h(j, _):
            hb_v[pl.ds(16 * j, 16)] = nsplat
            return 0
        lax.fori_loop(0, nslot // 16, fill_h, 0)
        pltpu.sync_copy(hb_v.at[pl.ds(0, nslot)],
                        head_sh.at[pl.ds(nslot * t, nslot)])

        plsc.subcore_barrier()

        r0 = s0 * SUB

        @pl.when(t > 0)
        def _():
            pltpu.sync_copy(batch_hbm.at[pl.ds(r0 - 16, 16)], prev_v)
        pvv = prev_v[pl.ds(0, 16)]
        pv_s[0] = jnp.where(t > 0, pvv[15], -1)
        cur_s[0] = -1

        def reset_acc():
            for d in range(DV):
                acc_v[pl.ds(16 * d, 16)] = minf16

        reset_acc()

        def flush():
            cur = cur_s[0]
            @pl.when(cur == fs_s[0])
            def _():
                pltpu.sync_copy(acc_v, emax_sh.at[2 * t])
            @pl.when(cur != fs_s[0])
            def _():
                pltpu.sync_copy(acc_v, maxout_sh.at[cur])

        def headwrite(sid, g):
            stage_v[pl.ds(0, 16)] = jnp.full((16,), 1, jnp.int32) * g
            pltpu.sync_copy(stage_v.at[pl.ds(0, 8)],
                            head_sh.at[pl.ds(sid * 8, 8)])

        def boundary(sid, g):
            cur = cur_s[0]
            @pl.when(cur >= 0)
            def _():
                flush()
            prev_row = jnp.where(cur >= 0, cur, pv_s[0])
            @pl.when(sid != prev_row)
            def _():
                headwrite(sid, g)
            cur_s[0] = sid

        def blockmax(eb, k):
            out = []
            for d in range(DV):
                v = [eb[16 * k + r, pl.ds(16 * d, 16)] for r in range(16)]
                while len(v) > 1:
                    v = [jnp.maximum(v[i], v[i + 1])
                         for i in range(0, len(v), 2)]
                out.append(v[0])
            return out

        def process_subchunk(row0, eb, ib):
            def block(k, _):
                iv = ib[pl.ds(16 * k, 16)]
                mn = iv[0]
                mx = iv[15]
                g0 = row0 + 16 * k
                uni = mn == mx

                @pl.when(jnp.logical_and(uni, mn != cur_s[0]))
                def _():
                    boundary(mn, g0)
                    reset_acc()

                @pl.when(uni)
                def _():
                    bm = blockmax(eb, k)
                    for d in range(DV):
                        acc_v[pl.ds(16 * d, 16)] = jnp.maximum(
                            acc_v[pl.ds(16 * d, 16)], bm[d])

                @pl.when(jnp.logical_not(uni))
                def _():
                    a = [acc_v[pl.ds(16 * d, 16)] for d in range(DV)]
                    for r in range(16):
                        sid = iv[r]
                        nb = sid != cur_s[0]
                        @pl.when(nb)
                        def _(a=a, sid=sid, r=r):
                            for d in range(DV):
                                acc_v[pl.ds(16 * d, 16)] = a[d]
                            boundary(sid, g0 + r)
                        row = [eb[16 * k + r, pl.ds(16 * d, 16)]
                               for d in range(DV)]
                        a = [jnp.where(nb, row[d],
                                       jnp.maximum(a[d], row[d]))
                             for d in range(DV)]
                    for d in range(DV):
                        acc_v[pl.ds(16 * d, 16)] = a[d]
                return 0

            lax.fori_loop(0, BLOCKS, block, 0)

        MAXP = (NSUB // NS + 2) // 2

        def pair(p, _):
            for b in range(2):
                si = 2 * p + b
                eb, ib, se, sj = bufs[b]
                @pl.when(si < nsub)
                def _(si=si, eb=eb, ib=ib, se=se, sj=sj):
                    for cp in dma_pair(si, eb, ib, se, sj):
                        cp.wait()
                    @pl.when(si == 0)
                    def _():
                        fs_s[0] = ib[pl.ds(0, 16)][0]
                    process_subchunk((s0 + si) * SUB, eb, ib)
                    @pl.when(si + 2 < nsub)
                    def _():
                        for cp in dma_pair(si + 2, eb, ib, se, sj):
                            cp.start()
            return 0

        lax.fori_loop(0, MAXP, pair, 0)

        stage_v[pl.ds(0, 16)] = jnp.full((16,), 1, jnp.int32) * fs_s[0]
        pltpu.sync_copy(stage_v.at[pl.ds(0, 8)],
                        eseg_sh.at[pl.ds(16 * t, 8)])
        stage_v[pl.ds(0, 16)] = jnp.full((16,), 1, jnp.int32) * cur_s[0]
        pltpu.sync_copy(stage_v.at[pl.ds(0, 8)],
                        eseg_sh.at[pl.ds(16 * t + 8, 8)])

        @pl.when(cur_s[0] == fs_s[0])
        def _():
            pltpu.sync_copy(acc_v, emax_sh.at[2 * t])
            pltpu.sync_copy(acc_v, emax_sh.at[2 * t + 1])

        @pl.when(cur_s[0] != fs_s[0])
        def _():
            pltpu.sync_copy(acc_v, emax_sh.at[2 * t + 1])

        plsc.subcore_barrier()

        @pl.when(t == 0)
        def _():
            pltpu.sync_copy(eseg_sh, eseg_v)
            pltpu.sync_copy(emax_sh, emax_v)

            iota16 = lax.iota(jnp.int32, 16)
            es0 = plsc.load_gather(eseg_v, [iota16 * 8])
            es1 = plsc.load_gather(eseg_v, [(iota16 + 16) * 8])

            cur_s[0] = es0[0]
            for d in range(DV):
                acc_v[pl.ds(16 * d, 16)] = emax_v[0, pl.ds(16 * d, 16)]

            for e in range(1, 2 * NS):
                s = es0[e] if e < 16 else es1[e - 16]
                @pl.when(s == cur_s[0])
                def _(s=s, e=e):
                    for d in range(DV):
                        acc_v[pl.ds(16 * d, 16)] = jnp.maximum(
                            acc_v[pl.ds(16 * d, 16)],
                            emax_v[e, pl.ds(16 * d, 16)])
                @pl.when(s != cur_s[0])
                def _(s=s, e=e):
                    pltpu.sync_copy(acc_v, maxout_sh.at[cur_s[0]])
                    cur_s[0] = s
                    for d in range(DV):
                        acc_v[pl.ds(16 * d, 16)] = emax_v[e, pl.ds(16 * d, 16)]
            pltpu.sync_copy(acc_v, maxout_sh.at[cur_s[0]])

            pltpu.sync_copy(head_sh, hb_v)
            iota16 = lax.iota(jnp.int32, 16)

            def bf(i, carry):
                ci = B // 16 - 1 - i
                idx = (ci * 16 + iota16) * 8
                v = plsc.load_gather(hb_v, [idx])
                r = lax.rev(-v, dimensions=(0,))
                c = jnp.maximum(plsc.cummax(r), carry)
                hc_v[pl.ds(ci * 16, 16)] = -lax.rev(c, dimensions=(0,))
                return c[15]
            lax.fori_loop(0, B // 16, bf, jnp.int32(-N))

            def gi(i, _):
                v = hc_v[pl.ds(16 * i, 16)]
                hg_v[pl.ds(16 * i, 16)] = jnp.minimum(v, N - 1)
                tg_v[pl.ds(16 * i, 16)] = jnp.minimum(v + 1, N - 1)
                return 0
            lax.fori_loop(0, B // 16, gi, 0)
            pltpu.sync_copy(hg_v, hg_sh)
            pltpu.sync_copy(tg_v, tg_sh)

        plsc.subcore_barrier()

        pltpu.sync_copy(hg_sh.at[pl.ds(GB * t, GB)], idx_v)
        pltpu.async_copy(emb_hbm.at[idx_v], row_v, sem).wait()
        pltpu.sync_copy(row_v, hout.at[pl.ds(GB * t, GB)])

        pltpu.sync_copy(tg_sh.at[pl.ds(GB * t, GB)], idx_v)
        pltpu.async_copy(emb_hbm.at[idx_v], row_v, sem).wait()
        pltpu.sync_copy(row_v, tout.at[pl.ds(GB * t, GB)])

        pltpu.sync_copy(maxout_sh.at[pl.ds(GB * t, GB)],
                        mout.at[pl.ds(GB * t, GB)])

    return sck(all_node_emb, graph_batch)


def _tc_proj(hrows, trows, mrows, W, b2):
    B, D = hrows.shape

    def body(h_ref, t_ref, m_ref, w_ref, b_ref, o_ref):
        w = w_ref[...]
        dn = (((1,), (1,)), ((), ()))
        acc = lax.dot_general(h_ref[...], w[:, 0:D], dn,
                              preferred_element_type=jnp.float32)
        acc += lax.dot_general(t_ref[...], w[:, D:2 * D], dn,
                               preferred_element_type=jnp.float32)
        acc += lax.dot_general(m_ref[...], w[:, 2 * D:3 * D], dn,
                               preferred_element_type=jnp.float32)
        o_ref[...] = acc + b_ref[...]

    return pl.pallas_call(
        body,
        out_shape=jax.ShapeDtypeStruct((B, D), jnp.float32),
    )(hrows, trows, mrows, W, b2)


def kernel(all_node_emb, supernode_edge_index, supernode_idx, graph_batch, W, b):
    B = supernode_idx.shape[0]
    D = all_node_emb.shape[1]
    hrows, trows, mrows = _sc_segment(all_node_emb, graph_batch, B)
    return _tc_proj(hrows, trows, mrows, W, b.reshape(1, D))


# speedup vs baseline: 5.5540x; 1.3669x over previous
import functools

import jax
import jax.numpy as jnp
from jax import lax
from jax.experimental import pallas as pl
from jax.experimental.pallas import tpu as pltpu
from jax.experimental.pallas import tpu_sc as plsc

NEG_INF = float("-inf")


def _sc_scan(all_node_emb, graph_batch, B):
    N, D = all_node_emb.shape
    NS = 16
    NC = 2
    NW = NC * NS
    SUB = 160
    NSUB = N // SUB
    BLOCKS = SUB // 16
    DV = D // 16
    GB = B // NS

    mesh = plsc.VectorSubcoreMesh(core_axis_name="c", subcore_axis_name="s")

    out_type = (
        jax.ShapeDtypeStruct((NC, B, D), jnp.float32),
        jax.ShapeDtypeStruct((NC, 8 * B), jnp.int32),
        jax.ShapeDtypeStruct((NC * 2 * 8,), jnp.int32),
        jax.ShapeDtypeStruct((NC, 2, D), jnp.float32),
    )

    scratch_types = [
        pltpu.VMEM((SUB, D), jnp.float32),
        pltpu.VMEM((SUB, D), jnp.float32),
        pltpu.VMEM((SUB,), jnp.int32),
        pltpu.VMEM((SUB,), jnp.int32),
        pltpu.SemaphoreType.DMA,
        pltpu.SemaphoreType.DMA,
        pltpu.SemaphoreType.DMA,
        pltpu.SemaphoreType.DMA,
        pltpu.VMEM((16,), jnp.int32),
        pltpu.VMEM((D,), jnp.float32),
        pltpu.VMEM((16,), jnp.int32),
        pltpu.VMEM((GB, D), jnp.float32),
        pltpu.VMEM((512,), jnp.int32),
        pltpu.VMEM((16 * NS,), jnp.int32),
        pltpu.VMEM((2 * NS, D), jnp.float32),
        pltpu.SMEM((1,), jnp.int32),
        pltpu.SMEM((1,), jnp.int32),
        pltpu.SMEM((1,), jnp.int32),
        pltpu.VMEM_SHARED((B, D), jnp.float32),
        pltpu.VMEM_SHARED((8 * B,), jnp.int32),
        pltpu.VMEM_SHARED((16 * NS,), jnp.int32),
        pltpu.VMEM_SHARED((2 * NS, D), jnp.float32),
    ]

    @functools.partial(
        pl.kernel, out_type=out_type, mesh=mesh, scratch_types=scratch_types,
        compiler_params=pltpu.CompilerParams(needs_layout_passes=False))
    def sck(emb_hbm, batch_hbm, mout, hout, csout, crout,
            emb0_v, emb1_v, ids0_v, ids1_v, seme0, seme1, semi0, semi1,
            prev_v, acc_v, stage_v, row_v, hfill_v, eseg_v, emax_v,
            cur_s, fs_s, pv_s,
            maxout_sh, head_sh, eseg_sh, emax_sh):
        c = lax.axis_index("c")
        s = lax.axis_index("s")
        w = c * NS + s

        minf16 = jnp.full((16,), NEG_INF, jnp.float32)
        bufs = ((emb0_v, ids0_v, seme0, semi0), (emb1_v, ids1_v, seme1, semi1))

        s0 = (w * NSUB) // NW
        s1 = ((w + 1) * NSUB) // NW
        nsub = s1 - s0
        r0 = s0 * SUB

        def dma_pair(si, eb, ib, se, sj):
            row0 = (s0 + si) * SUB
            return (pltpu.make_async_copy(emb_hbm.at[pl.ds(row0, SUB)], eb, se),
                    pltpu.make_async_copy(batch_hbm.at[pl.ds(row0, SUB)], ib, sj))

        for b in range(2):
            for cp in dma_pair(b, *bufs[b]):
                cp.start()

        def fill_row(j, _):
            for d in range(DV):
                row_v[j, pl.ds(16 * d, 16)] = minf16
            return 0
        lax.fori_loop(0, GB, fill_row, 0)
        pltpu.sync_copy(row_v, maxout_sh.at[pl.ds(GB * s, GB)])

        nslot = 8 * B // NS
        nsplat = jnp.full((16,), N, jnp.int32)
        def fill_h(j, _):
            hfill_v[pl.ds(16 * j, 16)] = nsplat
            return 0
        lax.fori_loop(0, nslot // 16, fill_h, 0)
        pltpu.sync_copy(hfill_v, head_sh.at[pl.ds(nslot * s, nslot)])

        plsc.subcore_barrier()

        @pl.when(w > 0)
        def _():
            pltpu.sync_copy(batch_hbm.at[pl.ds(r0 - 16, 16)], prev_v)
        pvv = prev_v[pl.ds(0, 16)]
        pv_s[0] = jnp.where(w > 0, pvv[15], -1)
        cur_s[0] = -1

        def reset_acc():
            for d in range(DV):
                acc_v[pl.ds(16 * d, 16)] = minf16

        reset_acc()

        def flush():
            cur = cur_s[0]
            @pl.when(cur == fs_s[0])
            def _():
                pltpu.sync_copy(acc_v, emax_sh.at[2 * s])
            @pl.when(cur != fs_s[0])
            def _():
                pltpu.sync_copy(acc_v, maxout_sh.at[cur])

        def headwrite(sid, g):
            stage_v[pl.ds(0, 16)] = jnp.full((16,), 1, jnp.int32) * g
            pltpu.sync_copy(stage_v.at[pl.ds(0, 8)],
                            head_sh.at[pl.ds(sid * 8, 8)])

        def boundary(sid, g):
            cur = cur_s[0]
            @pl.when(cur >= 0)
            def _():
                flush()
            prev_row = jnp.where(cur >= 0, cur, pv_s[0])
            @pl.when(sid != prev_row)
            def _():
                headwrite(sid, g)
            cur_s[0] = sid

        def blockmax(eb, k):
            out = []
            for d in range(DV):
                v = [eb[16 * k + r, pl.ds(16 * d, 16)] for r in range(16)]
                while len(v) > 1:
                    v = [jnp.maximum(v[i], v[i + 1])
                         for i in range(0, len(v), 2)]
                out.append(v[0])
            return out

        def process_subchunk(row0, eb, ib):
            def block(k, _):
                iv = ib[pl.ds(16 * k, 16)]
                mn = iv[0]
                mx = iv[15]
                g0 = row0 + 16 * k
                uni = mn == mx

                @pl.when(jnp.logical_and(uni, mn != cur_s[0]))
                def _():
                    boundary(mn, g0)
                    reset_acc()

                @pl.when(uni)
                def _():
                    bm = blockmax(eb, k)
                    for d in range(DV):
                        acc_v[pl.ds(16 * d, 16)] = jnp.maximum(
                            acc_v[pl.ds(16 * d, 16)], bm[d])

                @pl.when(jnp.logical_not(uni))
                def _():
                    a = [acc_v[pl.ds(16 * d, 16)] for d in range(DV)]
                    for r in range(16):
                        sid = iv[r]
                        nb = sid != cur_s[0]
                        @pl.when(nb)
                        def _(a=a, sid=sid, r=r):
                            for d in range(DV):
                                acc_v[pl.ds(16 * d, 16)] = a[d]
                            boundary(sid, g0 + r)
                        row = [eb[16 * k + r, pl.ds(16 * d, 16)]
                               for d in range(DV)]
                        a = [jnp.where(nb, row[d],
                                       jnp.maximum(a[d], row[d]))
                             for d in range(DV)]
                    for d in range(DV):
                        acc_v[pl.ds(16 * d, 16)] = a[d]
                return 0

            lax.fori_loop(0, BLOCKS, block, 0)

        MAXP = (NSUB // NW + 2) // 2

        def pair(p, _):
            for b in range(2):
                si = 2 * p + b
                eb, ib, se, sj = bufs[b]
                @pl.when(si < nsub)
                def _(si=si, eb=eb, ib=ib, se=se, sj=sj):
                    for cp in dma_pair(si, eb, ib, se, sj):
                        cp.wait()
                    @pl.when(si == 0)
                    def _():
                        fs_s[0] = ib[pl.ds(0, 16)][0]
                    process_subchunk((s0 + si) * SUB, eb, ib)
                    @pl.when(si + 2 < nsub)
                    def _():
                        for cp in dma_pair(si + 2, eb, ib, se, sj):
                            cp.start()
            return 0

        lax.fori_loop(0, MAXP, pair, 0)

        stage_v[pl.ds(0, 16)] = jnp.full((16,), 1, jnp.int32) * fs_s[0]
        pltpu.sync_copy(stage_v.at[pl.ds(0, 8)],
                        eseg_sh.at[pl.ds(16 * s, 8)])
        stage_v[pl.ds(0, 16)] = jnp.full((16,), 1, jnp.int32) * cur_s[0]
        pltpu.sync_copy(stage_v.at[pl.ds(0, 8)],
                        eseg_sh.at[pl.ds(16 * s + 8, 8)])

        @pl.when(cur_s[0] == fs_s[0])
        def _():
            pltpu.sync_copy(acc_v, emax_sh.at[2 * s])
            pltpu.sync_copy(acc_v, emax_sh.at[2 * s + 1])

        @pl.when(cur_s[0] != fs_s[0])
        def _():
            pltpu.sync_copy(acc_v, emax_sh.at[2 * s + 1])

        plsc.subcore_barrier()

        @pl.when(s == 0)
        def _():
            pltpu.sync_copy(eseg_sh, eseg_v)
            pltpu.sync_copy(emax_sh, emax_v)

            iota16 = lax.iota(jnp.int32, 16)
            es0 = plsc.load_gather(eseg_v, [iota16 * 8])
            es1 = plsc.load_gather(eseg_v, [(iota16 + 16) * 8])
            first = es0[0]

            cur_s[0] = first
            for d in range(DV):
                acc_v[pl.ds(16 * d, 16)] = emax_v[0, pl.ds(16 * d, 16)]

            def cflush():
                cur = cur_s[0]
                @pl.when(cur == first)
                def _():
                    pltpu.sync_copy(acc_v, crout.at[c, 0])
                @pl.when(cur != first)
                def _():
                    pltpu.sync_copy(acc_v, maxout_sh.at[cur])

            for e in range(1, 2 * NS):
                sid = es0[e] if e < 16 else es1[e - 16]
                @pl.when(sid == cur_s[0])
                def _(sid=sid, e=e):
                    for d in range(DV):
                        acc_v[pl.ds(16 * d, 16)] = jnp.maximum(
                            acc_v[pl.ds(16 * d, 16)],
                            emax_v[e, pl.ds(16 * d, 16)])
                @pl.when(sid != cur_s[0])
                def _(sid=sid, e=e):
                    cflush()
                    cur_s[0] = sid
                    for d in range(DV):
                        acc_v[pl.ds(16 * d, 16)] = emax_v[e, pl.ds(16 * d, 16)]

            @pl.when(cur_s[0] == first)
            def _():
                pltpu.sync_copy(acc_v, crout.at[c, 0])
            pltpu.sync_copy(acc_v, crout.at[c, 1])

            stage_v[pl.ds(0, 16)] = jnp.full((16,), 1, jnp.int32) * first
            pltpu.sync_copy(stage_v.at[pl.ds(0, 8)],
                            csout.at[pl.ds(c * 16, 8)])
            stage_v[pl.ds(0, 16)] = jnp.full((16,), 1, jnp.int32) * cur_s[0]
            pltpu.sync_copy(stage_v.at[pl.ds(0, 8)],
                            csout.at[pl.ds(c * 16 + 8, 8)])

            pltpu.sync_copy(head_sh, hout.at[c])

        plsc.subcore_barrier()

        pltpu.sync_copy(maxout_sh.at[pl.ds(GB * s, GB)],
                        mout.at[c, pl.ds(GB * s, GB)])

    return sck(all_node_emb, graph_batch)


def _sc_gather(all_node_emb, headout, B):
    N, D = all_node_emb.shape
    NS = 16
    GB = B // NS

    mesh = plsc.VectorSubcoreMesh(
        core_axis_name="c", subcore_axis_name="s", num_cores=1)

    out_type = (
        jax.ShapeDtypeStruct((B, D), jnp.float32),
        jax.ShapeDtypeStruct((B, D), jnp.float32),
    )

    scratch_types = [
        pltpu.VMEM((8 * B,), jnp.int32),
        pltpu.VMEM((8 * B,), jnp.int32),
        pltpu.VMEM((B,), jnp.int32),
        pltpu.VMEM((B,), jnp.int32),
        pltpu.VMEM((B,), jnp.int32),
        pltpu.VMEM((GB,), jnp.int32),
        pltpu.VMEM((GB, D), jnp.float32),
        pltpu.SemaphoreType.DMA,
        pltpu.VMEM_SHARED((B,), jnp.int32),
        pltpu.VMEM_SHARED((B,), jnp.int32),
    ]

    @functools.partial(
        pl.kernel, out_type=out_type, mesh=mesh, scratch_types=scratch_types,
        compiler_params=pltpu.CompilerParams(needs_layout_passes=False))
    def gk(emb_hbm, hin, hout, tout,
           h0_v, h1_v, hc_v, hg_v, tg_v, idx_v, row_v, sem,
           hg_sh, tg_sh):
        t = lax.axis_index("s")

        @pl.when(t == 0)
        def _():
            pltpu.sync_copy(hin.at[0], h0_v)
            pltpu.sync_copy(hin.at[1], h1_v)

            def mn(j, _):
                h0_v[pl.ds(16 * j, 16)] = jnp.minimum(
                    h0_v[pl.ds(16 * j, 16)], h1_v[pl.ds(16 * j, 16)])
                return 0
            lax.fori_loop(0, 8 * B // 16, mn, 0)

            iota16 = lax.iota(jnp.int32, 16)

            def bf(i, carry):
                ci = B // 16 - 1 - i
                idx = (ci * 16 + iota16) * 8
                v = plsc.load_gather(h0_v, [idx])
                r = lax.rev(-v, dimensions=(0,))
                cc = jnp.maximum(plsc.cummax(r), carry)
                hc_v[pl.ds(ci * 16, 16)] = -lax.rev(cc, dimensions=(0,))
                return cc[15]
            lax.fori_loop(0, B // 16, bf, jnp.int32(-N))

            def gi(i, _):
                v = hc_v[pl.ds(16 * i, 16)]
                hg_v[pl.ds(16 * i, 16)] = jnp.minimum(v, N - 1)
                tg_v[pl.ds(16 * i, 16)] = jnp.minimum(v + 1, N - 1)
                return 0
            lax.fori_loop(0, B // 16, gi, 0)
            pltpu.sync_copy(hg_v, hg_sh)
            pltpu.sync_copy(tg_v, tg_sh)

        plsc.subcore_barrier()

        pltpu.sync_copy(hg_sh.at[pl.ds(GB * t, GB)], idx_v)
        pltpu.async_copy(emb_hbm.at[idx_v], row_v, sem).wait()
        pltpu.sync_copy(row_v, hout.at[pl.ds(GB * t, GB)])

        pltpu.sync_copy(tg_sh.at[pl.ds(GB * t, GB)], idx_v)
        pltpu.async_copy(emb_hbm.at[idx_v], row_v, sem).wait()
        pltpu.sync_copy(row_v, tout.at[pl.ds(GB * t, GB)])

    return gk(all_node_emb, headout)


def _tc_proj(hrows, trows, maxout2, cseg_f, crow_f, W, b2):
    B, D = hrows.shape

    def body(h_ref, t_ref, m_ref, s_ref, r_ref, w_ref, b_ref, o_ref):
        maxp = jnp.maximum(m_ref[0], m_ref[1])
        segs = s_ref[...]
        rows = r_ref[...]
        iota_b = lax.broadcasted_iota(jnp.int32, (B, 1), 0)
        for j in range(4):
            mask = iota_b == segs[j:j + 1, 0:1]
            maxp = jnp.where(mask, jnp.maximum(maxp, rows[j:j + 1, :]), maxp)

        w = w_ref[...]
        dn = (((1,), (1,)), ((), ()))
        acc = lax.dot_general(h_ref[...], w[:, 0:D], dn,
                              preferred_element_type=jnp.float32)
        acc += lax.dot_general(t_ref[...], w[:, D:2 * D], dn,
                               preferred_element_type=jnp.float32)
        acc += lax.dot_general(maxp, w[:, 2 * D:3 * D], dn,
                               preferred_element_type=jnp.float32)
        o_ref[...] = acc + b_ref[...]

    return pl.pallas_call(
        body,
        out_shape=jax.ShapeDtypeStruct((B, D), jnp.float32),
    )(hrows, trows, maxout2, cseg_f, crow_f, W, b2)


def kernel(all_node_emb, supernode_edge_index, supernode_idx, graph_batch, W, b):
    B = supernode_idx.shape[0]
    D = all_node_emb.shape[1]
    maxout2, headout, cseg, crow = _sc_scan(all_node_emb, graph_batch, B)
    hrows, trows = _sc_gather(all_node_emb, headout, B)
    return _tc_proj(hrows, trows, maxout2,
                    cseg.reshape(4, 8), crow.reshape(4, D),
                    W, b.reshape(1, D))


# speedup vs baseline: 5.5731x; 1.0034x over previous
import functools

import jax
import jax.numpy as jnp
from jax import lax
from jax.experimental import pallas as pl
from jax.experimental.pallas import tpu as pltpu
from jax.experimental.pallas import tpu_sc as plsc

NEG_INF = float("-inf")


def _sc_scan(all_node_emb, graph_batch, B):
    N, D = all_node_emb.shape
    NS = 16
    NC = 2
    NW = NC * NS
    SUB = 400
    NSUB = N // SUB
    BLOCKS = SUB // 16
    DV = D // 16
    GB = B // NS

    mesh = plsc.VectorSubcoreMesh(core_axis_name="c", subcore_axis_name="s")

    out_type = (
        jax.ShapeDtypeStruct((NC, B, D), jnp.float32),
        jax.ShapeDtypeStruct((NC, 8 * B), jnp.int32),
        jax.ShapeDtypeStruct((NC * 2 * 8,), jnp.int32),
        jax.ShapeDtypeStruct((NC, 2, D), jnp.float32),
    )

    scratch_types = [
        pltpu.VMEM((SUB, D), jnp.float32),
        pltpu.VMEM((SUB, D), jnp.float32),
        pltpu.VMEM((SUB,), jnp.int32),
        pltpu.VMEM((SUB,), jnp.int32),
        pltpu.SemaphoreType.DMA,
        pltpu.SemaphoreType.DMA,
        pltpu.SemaphoreType.DMA,
        pltpu.SemaphoreType.DMA,
        pltpu.VMEM((16,), jnp.int32),
        pltpu.VMEM((D,), jnp.float32),
        pltpu.VMEM((16,), jnp.int32),
        pltpu.VMEM((GB, D), jnp.float32),
        pltpu.VMEM((512,), jnp.int32),
        pltpu.VMEM((16 * NS,), jnp.int32),
        pltpu.VMEM((2 * NS, D), jnp.float32),
        pltpu.SMEM((1,), jnp.int32),
        pltpu.SMEM((1,), jnp.int32),
        pltpu.SMEM((1,), jnp.int32),
        pltpu.VMEM_SHARED((B, D), jnp.float32),
        pltpu.VMEM_SHARED((8 * B,), jnp.int32),
        pltpu.VMEM_SHARED((16 * NS,), jnp.int32),
        pltpu.VMEM_SHARED((2 * NS, D), jnp.float32),
    ]

    @functools.partial(
        pl.kernel, out_type=out_type, mesh=mesh, scratch_types=scratch_types,
        compiler_params=pltpu.CompilerParams(needs_layout_passes=False))
    def sck(emb_hbm, batch_hbm, mout, hout, csout, crout,
            emb0_v, emb1_v, ids0_v, ids1_v, seme0, seme1, semi0, semi1,
            prev_v, acc_v, stage_v, row_v, hfill_v, eseg_v, emax_v,
            cur_s, fs_s, pv_s,
            maxout_sh, head_sh, eseg_sh, emax_sh):
        c = lax.axis_index("c")
        s = lax.axis_index("s")
        w = c * NS + s

        minf16 = jnp.full((16,), NEG_INF, jnp.float32)
        bufs = ((emb0_v, ids0_v, seme0, semi0), (emb1_v, ids1_v, seme1, semi1))

        s0 = (w * NSUB) // NW
        s1 = ((w + 1) * NSUB) // NW
        nsub = s1 - s0
        r0 = s0 * SUB

        def dma_pair(si, eb, ib, se, sj):
            row0 = (s0 + si) * SUB
            return (pltpu.make_async_copy(emb_hbm.at[pl.ds(row0, SUB)], eb, se),
                    pltpu.make_async_copy(batch_hbm.at[pl.ds(row0, SUB)], ib, sj))

        for b in range(2):
            for cp in dma_pair(b, *bufs[b]):
                cp.start()

        def fill_row(j, _):
            for d in range(DV):
                row_v[j, pl.ds(16 * d, 16)] = minf16
            return 0
        lax.fori_loop(0, GB, fill_row, 0)
        pltpu.sync_copy(row_v, maxout_sh.at[pl.ds(GB * s, GB)])

        nslot = 8 * B // NS
        nsplat = jnp.full((16,), N, jnp.int32)
        def fill_h(j, _):
            hfill_v[pl.ds(16 * j, 16)] = nsplat
            return 0
        lax.fori_loop(0, nslot // 16, fill_h, 0)
        pltpu.sync_copy(hfill_v, head_sh.at[pl.ds(nslot * s, nslot)])

        plsc.subcore_barrier()

        @pl.when(w > 0)
        def _():
            pltpu.sync_copy(batch_hbm.at[pl.ds(r0 - 16, 16)], prev_v)
        pvv = prev_v[pl.ds(0, 16)]
        pv_s[0] = jnp.where(w > 0, pvv[15], -1)
        cur_s[0] = -1

        def reset_acc():
            for d in range(DV):
                acc_v[pl.ds(16 * d, 16)] = minf16

        reset_acc()

        def flush():
            cur = cur_s[0]
            @pl.when(cur == fs_s[0])
            def _():
                pltpu.sync_copy(acc_v, emax_sh.at[2 * s])
            @pl.when(cur != fs_s[0])
            def _():
                pltpu.sync_copy(acc_v, maxout_sh.at[cur])

        def headwrite(sid, g):
            stage_v[pl.ds(0, 16)] = jnp.full((16,), 1, jnp.int32) * g
            pltpu.sync_copy(stage_v.at[pl.ds(0, 8)],
                            head_sh.at[pl.ds(sid * 8, 8)])

        def boundary(sid, g):
            cur = cur_s[0]
            @pl.when(cur >= 0)
            def _():
                flush()
            prev_row = jnp.where(cur >= 0, cur, pv_s[0])
            @pl.when(sid != prev_row)
            def _():
                headwrite(sid, g)
            cur_s[0] = sid

        def blockmax(eb, k):
            out = []
            for d in range(DV):
                v = [eb[16 * k + r, pl.ds(16 * d, 16)] for r in range(16)]
                while len(v) > 1:
                    v = [jnp.maximum(v[i], v[i + 1])
                         for i in range(0, len(v), 2)]
                out.append(v[0])
            return out

        def process_subchunk(row0, eb, ib):
            def block(k, _):
                iv = ib[pl.ds(16 * k, 16)]
                mn = iv[0]
                mx = iv[15]
                g0 = row0 + 16 * k
                uni = mn == mx

                @pl.when(jnp.logical_and(uni, mn != cur_s[0]))
                def _():
                    boundary(mn, g0)
                    reset_acc()

                @pl.when(uni)
                def _():
                    bm = blockmax(eb, k)
                    for d in range(DV):
                        acc_v[pl.ds(16 * d, 16)] = jnp.maximum(
                            acc_v[pl.ds(16 * d, 16)], bm[d])

                @pl.when(jnp.logical_not(uni))
                def _():
                    a = [acc_v[pl.ds(16 * d, 16)] for d in range(DV)]
                    for r in range(16):
                        sid = iv[r]
                        nb = sid != cur_s[0]
                        @pl.when(nb)
                        def _(a=a, sid=sid, r=r):
                            for d in range(DV):
                                acc_v[pl.ds(16 * d, 16)] = a[d]
                            boundary(sid, g0 + r)
                        row = [eb[16 * k + r, pl.ds(16 * d, 16)]
                               for d in range(DV)]
                        a = [jnp.where(nb, row[d],
                                       jnp.maximum(a[d], row[d]))
                             for d in range(DV)]
                    for d in range(DV):
                        acc_v[pl.ds(16 * d, 16)] = a[d]
                return 0

            lax.fori_loop(0, BLOCKS, block, 0)

        MAXP = (NSUB // NW + 2) // 2

        def pair(p, _):
            for b in range(2):
                si = 2 * p + b
                eb, ib, se, sj = bufs[b]
                @pl.when(si < nsub)
                def _(si=si, eb=eb, ib=ib, se=se, sj=sj):
                    for cp in dma_pair(si, eb, ib, se, sj):
                        cp.wait()
                    @pl.when(si == 0)
                    def _():
                        fs_s[0] = ib[pl.ds(0, 16)][0]
                    process_subchunk((s0 + si) * SUB, eb, ib)
                    @pl.when(si + 2 < nsub)
                    def _():
                        for cp in dma_pair(si + 2, eb, ib, se, sj):
                            cp.start()
            return 0

        lax.fori_loop(0, MAXP, pair, 0)

        stage_v[pl.ds(0, 16)] = jnp.full((16,), 1, jnp.int32) * fs_s[0]
        pltpu.sync_copy(stage_v.at[pl.ds(0, 8)],
                        eseg_sh.at[pl.ds(16 * s, 8)])
        stage_v[pl.ds(0, 16)] = jnp.full((16,), 1, jnp.int32) * cur_s[0]
        pltpu.sync_copy(stage_v.at[pl.ds(0, 8)],
                        eseg_sh.at[pl.ds(16 * s + 8, 8)])

        @pl.when(cur_s[0] == fs_s[0])
        def _():
            pltpu.sync_copy(acc_v, emax_sh.at[2 * s])
            pltpu.sync_copy(acc_v, emax_sh.at[2 * s + 1])

        @pl.when(cur_s[0] != fs_s[0])
        def _():
            pltpu.sync_copy(acc_v, emax_sh.at[2 * s + 1])

        plsc.subcore_barrier()

        @pl.when(s == 0)
        def _():
            pltpu.sync_copy(eseg_sh, eseg_v)
            pltpu.sync_copy(emax_sh, emax_v)

            iota16 = lax.iota(jnp.int32, 16)
            es0 = plsc.load_gather(eseg_v, [iota16 * 8])
            es1 = plsc.load_gather(eseg_v, [(iota16 + 16) * 8])
            first = es0[0]

            cur_s[0] = first
            for d in range(DV):
                acc_v[pl.ds(16 * d, 16)] = emax_v[0, pl.ds(16 * d, 16)]

            def cflush():
                cur = cur_s[0]
                @pl.when(cur == first)
                def _():
                    pltpu.sync_copy(acc_v, crout.at[c, 0])
                @pl.when(cur != first)
                def _():
                    pltpu.sync_copy(acc_v, maxout_sh.at[cur])

            for e in range(1, 2 * NS):
                sid = es0[e] if e < 16 else es1[e - 16]
                @pl.when(sid == cur_s[0])
                def _(sid=sid, e=e):
                    for d in range(DV):
                        acc_v[pl.ds(16 * d, 16)] = jnp.maximum(
                            acc_v[pl.ds(16 * d, 16)],
                            emax_v[e, pl.ds(16 * d, 16)])
                @pl.when(sid != cur_s[0])
                def _(sid=sid, e=e):
                    cflush()
                    cur_s[0] = sid
                    for d in range(DV):
                        acc_v[pl.ds(16 * d, 16)] = emax_v[e, pl.ds(16 * d, 16)]

            @pl.when(cur_s[0] == first)
            def _():
                pltpu.sync_copy(acc_v, crout.at[c, 0])
            pltpu.sync_copy(acc_v, crout.at[c, 1])

            stage_v[pl.ds(0, 16)] = jnp.full((16,), 1, jnp.int32) * first
            pltpu.sync_copy(stage_v.at[pl.ds(0, 8)],
                            csout.at[pl.ds(c * 16, 8)])
            stage_v[pl.ds(0, 16)] = jnp.full((16,), 1, jnp.int32) * cur_s[0]
            pltpu.sync_copy(stage_v.at[pl.ds(0, 8)],
                            csout.at[pl.ds(c * 16 + 8, 8)])

            pltpu.sync_copy(head_sh, hout.at[c])

        plsc.subcore_barrier()

        pltpu.sync_copy(maxout_sh.at[pl.ds(GB * s, GB)],
                        mout.at[c, pl.ds(GB * s, GB)])

    return sck(all_node_emb, graph_batch)


def _sc_gather(all_node_emb, headout, B):
    N, D = all_node_emb.shape
    NS = 16
    GB = B // NS

    mesh = plsc.VectorSubcoreMesh(
        core_axis_name="c", subcore_axis_name="s", num_cores=1)

    out_type = (
        jax.ShapeDtypeStruct((B, D), jnp.float32),
        jax.ShapeDtypeStruct((B, D), jnp.float32),
    )

    scratch_types = [
        pltpu.VMEM((8 * B,), jnp.int32),
        pltpu.VMEM((8 * B,), jnp.int32),
        pltpu.VMEM((B,), jnp.int32),
        pltpu.VMEM((B,), jnp.int32),
        pltpu.VMEM((B,), jnp.int32),
        pltpu.VMEM((GB,), jnp.int32),
        pltpu.VMEM((GB, D), jnp.float32),
        pltpu.SemaphoreType.DMA,
        pltpu.VMEM_SHARED((B,), jnp.int32),
        pltpu.VMEM_SHARED((B,), jnp.int32),
    ]

    @functools.partial(
        pl.kernel, out_type=out_type, mesh=mesh, scratch_types=scratch_types,
        compiler_params=pltpu.CompilerParams(needs_layout_passes=False))
    def gk(emb_hbm, hin, hout, tout,
           h0_v, h1_v, hc_v, hg_v, tg_v, idx_v, row_v, sem,
           hg_sh, tg_sh):
        t = lax.axis_index("s")

        @pl.when(t == 0)
        def _():
            pltpu.sync_copy(hin.at[0], h0_v)
            pltpu.sync_copy(hin.at[1], h1_v)

            def mn(j, _):
                h0_v[pl.ds(16 * j, 16)] = jnp.minimum(
                    h0_v[pl.ds(16 * j, 16)], h1_v[pl.ds(16 * j, 16)])
                return 0
            lax.fori_loop(0, 8 * B // 16, mn, 0)

            iota16 = lax.iota(jnp.int32, 16)

            def bf(i, carry):
                ci = B // 16 - 1 - i
                idx = (ci * 16 + iota16) * 8
                v = plsc.load_gather(h0_v, [idx])
                r = lax.rev(-v, dimensions=(0,))
                cc = jnp.maximum(plsc.cummax(r), carry)
                hc_v[pl.ds(ci * 16, 16)] = -lax.rev(cc, dimensions=(0,))
                return cc[15]
            lax.fori_loop(0, B // 16, bf, jnp.int32(-N))

            def gi(i, _):
                v = hc_v[pl.ds(16 * i, 16)]
                hg_v[pl.ds(16 * i, 16)] = jnp.minimum(v, N - 1)
                tg_v[pl.ds(16 * i, 16)] = jnp.minimum(v + 1, N - 1)
                return 0
            lax.fori_loop(0, B // 16, gi, 0)
            pltpu.sync_copy(hg_v, hg_sh)
            pltpu.sync_copy(tg_v, tg_sh)

        plsc.subcore_barrier()

        pltpu.sync_copy(hg_sh.at[pl.ds(GB * t, GB)], idx_v)
        pltpu.async_copy(emb_hbm.at[idx_v], row_v, sem).wait()
        pltpu.sync_copy(row_v, hout.at[pl.ds(GB * t, GB)])

        pltpu.sync_copy(tg_sh.at[pl.ds(GB * t, GB)], idx_v)
        pltpu.async_copy(emb_hbm.at[idx_v], row_v, sem).wait()
        pltpu.sync_copy(row_v, tout.at[pl.ds(GB * t, GB)])

    return gk(all_node_emb, headout)


def _tc_proj(hrows, trows, maxout2, cseg_f, crow_f, W, b2):
    B, D = hrows.shape

    def body(h_ref, t_ref, m_ref, s_ref, r_ref, w_ref, b_ref, o_ref):
        maxp = jnp.maximum(m_ref[0], m_ref[1])
        segs = s_ref[...]
        rows = r_ref[...]
        iota_b = lax.broadcasted_iota(jnp.int32, (B, 1), 0)
        for j in range(4):
            mask = iota_b == segs[j:j + 1, 0:1]
            maxp = jnp.where(mask, jnp.maximum(maxp, rows[j:j + 1, :]), maxp)

        w = w_ref[...]
        dn = (((1,), (1,)), ((), ()))
        acc = lax.dot_general(h_ref[...], w[:, 0:D], dn,
                              preferred_element_type=jnp.float32)
        acc += lax.dot_general(t_ref[...], w[:, D:2 * D], dn,
                               preferred_element_type=jnp.float32)
        acc += lax.dot_general(maxp, w[:, 2 * D:3 * D], dn,
                               preferred_element_type=jnp.float32)
        o_ref[...] = acc + b_ref[...]

    return pl.pallas_call(
        body,
        out_shape=jax.ShapeDtypeStruct((B, D), jnp.float32),
    )(hrows, trows, maxout2, cseg_f, crow_f, W, b2)


def kernel(all_node_emb, supernode_edge_index, supernode_idx, graph_batch, W, b):
    B = supernode_idx.shape[0]
    D = all_node_emb.shape[1]
    maxout2, headout, cseg, crow = _sc_scan(all_node_emb, graph_batch, B)
    hrows, trows = _sc_gather(all_node_emb, headout, B)
    return _tc_proj(hrows, trows, maxout2,
                    cseg.reshape(4, 8), crow.reshape(4, D),
                    W, b.reshape(1, D))


# speedup vs baseline: 5.6252x; 1.0093x over previous
import functools

import jax
import jax.numpy as jnp
from jax import lax
from jax.experimental import pallas as pl
from jax.experimental.pallas import tpu as pltpu
from jax.experimental.pallas import tpu_sc as plsc

NEG_INF = float("-inf")


def _sc_scan(all_node_emb, graph_batch, B):
    N, D = all_node_emb.shape
    NS = 16
    NC = 2
    NW = NC * NS
    SUB = 400
    NSUB = N // SUB
    BLOCKS = SUB // 16
    DV = D // 16
    GB = B // NS

    mesh = plsc.VectorSubcoreMesh(core_axis_name="c", subcore_axis_name="s")

    out_type = (
        jax.ShapeDtypeStruct((NC, B, D), jnp.float32),
        jax.ShapeDtypeStruct((NC, 8 * B), jnp.int32),
        jax.ShapeDtypeStruct((NC * 2 * 8,), jnp.int32),
        jax.ShapeDtypeStruct((NC, 2, D), jnp.float32),
    )

    scratch_types = [
        pltpu.VMEM((SUB, D), jnp.float32),
        pltpu.VMEM((SUB, D), jnp.float32),
        pltpu.VMEM((SUB,), jnp.int32),
        pltpu.VMEM((SUB,), jnp.int32),
        pltpu.SemaphoreType.DMA,
        pltpu.SemaphoreType.DMA,
        pltpu.SemaphoreType.DMA,
        pltpu.SemaphoreType.DMA,
        pltpu.VMEM((16,), jnp.int32),
        pltpu.VMEM((D,), jnp.float32),
        pltpu.VMEM((16,), jnp.int32),
        pltpu.VMEM((GB, D), jnp.float32),
        pltpu.VMEM((512,), jnp.int32),
        pltpu.VMEM((16 * NS,), jnp.int32),
        pltpu.VMEM((2 * NS, D), jnp.float32),
        pltpu.SMEM((1,), jnp.int32),
        pltpu.SMEM((1,), jnp.int32),
        pltpu.SMEM((1,), jnp.int32),
        pltpu.VMEM_SHARED((B, D), jnp.float32),
        pltpu.VMEM_SHARED((8 * B,), jnp.int32),
        pltpu.VMEM_SHARED((16 * NS,), jnp.int32),
        pltpu.VMEM_SHARED((2 * NS, D), jnp.float32),
    ]

    @functools.partial(
        pl.kernel, out_type=out_type, mesh=mesh, scratch_types=scratch_types,
        compiler_params=pltpu.CompilerParams(needs_layout_passes=False))
    def sck(emb_hbm, batch_hbm, mout, hout, csout, crout,
            emb0_v, emb1_v, ids0_v, ids1_v, seme0, seme1, semi0, semi1,
            prev_v, acc_v, stage_v, row_v, hfill_v, eseg_v, emax_v,
            cur_s, fs_s, pv_s,
            maxout_sh, head_sh, eseg_sh, emax_sh):
        c = lax.axis_index("c")
        s = lax.axis_index("s")
        w = c * NS + s

        minf16 = jnp.full((16,), NEG_INF, jnp.float32)
        bufs = ((emb0_v, ids0_v, seme0, semi0), (emb1_v, ids1_v, seme1, semi1))

        s0 = (w * NSUB) // NW
        s1 = ((w + 1) * NSUB) // NW
        nsub = s1 - s0
        r0 = s0 * SUB

        def dma_pair(si, eb, ib, se, sj):
            row0 = (s0 + si) * SUB
            return (pltpu.make_async_copy(emb_hbm.at[pl.ds(row0, SUB)], eb, se),
                    pltpu.make_async_copy(batch_hbm.at[pl.ds(row0, SUB)], ib, sj))

        for b in range(2):
            for cp in dma_pair(b, *bufs[b]):
                cp.start()

        def fill_row(j, _):
            for d in range(DV):
                row_v[j, pl.ds(16 * d, 16)] = minf16
            return 0
        lax.fori_loop(0, GB, fill_row, 0)
        pltpu.sync_copy(row_v, maxout_sh.at[pl.ds(GB * s, GB)])

        nslot = 8 * B // NS
        nsplat = jnp.full((16,), N, jnp.int32)
        def fill_h(j, _):
            hfill_v[pl.ds(16 * j, 16)] = nsplat
            return 0
        lax.fori_loop(0, nslot // 16, fill_h, 0)
        pltpu.sync_copy(hfill_v, head_sh.at[pl.ds(nslot * s, nslot)])

        plsc.subcore_barrier()

        @pl.when(w > 0)
        def _():
            pltpu.sync_copy(batch_hbm.at[pl.ds(r0 - 16, 16)], prev_v)
        pvv = prev_v[pl.ds(0, 16)]
        pv_s[0] = jnp.where(w > 0, pvv[15], -1)
        cur_s[0] = -1

        def reset_acc():
            for d in range(DV):
                acc_v[pl.ds(16 * d, 16)] = minf16

        reset_acc()

        def flush():
            cur = cur_s[0]
            @pl.when(cur == fs_s[0])
            def _():
                pltpu.sync_copy(acc_v, emax_sh.at[2 * s])
            @pl.when(cur != fs_s[0])
            def _():
                pltpu.sync_copy(acc_v, maxout_sh.at[cur])

        def headwrite(sid, g):
            stage_v[pl.ds(0, 16)] = jnp.full((16,), 1, jnp.int32) * g
            pltpu.sync_copy(stage_v.at[pl.ds(0, 8)],
                            head_sh.at[pl.ds(sid * 8, 8)])

        def boundary(sid, g):
            cur = cur_s[0]
            @pl.when(cur >= 0)
            def _():
                flush()
            prev_row = jnp.where(cur >= 0, cur, pv_s[0])
            @pl.when(sid != prev_row)
            def _():
                headwrite(sid, g)
            cur_s[0] = sid

        def blockmax(eb, k):
            out = []
            for d in range(DV):
                v = [eb[16 * k + r, pl.ds(16 * d, 16)] for r in range(16)]
                while len(v) > 1:
                    v = [jnp.maximum(v[i], v[i + 1])
                         for i in range(0, len(v), 2)]
                out.append(v[0])
            return out

        def process_subchunk(row0, eb, ib):
            def block(k, _):
                iv = ib[pl.ds(16 * k, 16)]
                mn = iv[0]
                mx = iv[15]
                g0 = row0 + 16 * k
                uni = mn == mx

                @pl.when(jnp.logical_and(uni, mn != cur_s[0]))
                def _():
                    boundary(mn, g0)
                    reset_acc()

                @pl.when(uni)
                def _():
                    bm = blockmax(eb, k)
                    for d in range(DV):
                        acc_v[pl.ds(16 * d, 16)] = jnp.maximum(
                            acc_v[pl.ds(16 * d, 16)], bm[d])

                @pl.when(jnp.logical_not(uni))
                def _():
                    a = [acc_v[pl.ds(16 * d, 16)] for d in range(DV)]
                    for r in range(16):
                        sid = iv[r]
                        nb = sid != cur_s[0]
                        @pl.when(nb)
                        def _(a=a, sid=sid, r=r):
                            for d in range(DV):
                                acc_v[pl.ds(16 * d, 16)] = a[d]
                            boundary(sid, g0 + r)
                        row = [eb[16 * k + r, pl.ds(16 * d, 16)]
                               for d in range(DV)]
                        a = [jnp.where(nb, row[d],
                                       jnp.maximum(a[d], row[d]))
                             for d in range(DV)]
                    for d in range(DV):
                        acc_v[pl.ds(16 * d, 16)] = a[d]
                return 0

            lax.fori_loop(0, BLOCKS, block, 0)

        MAXP = (NSUB // NW + 2) // 2

        def pair(p, _):
            for b in range(2):
                si = 2 * p + b
                eb, ib, se, sj = bufs[b]
                @pl.when(si < nsub)
                def _(si=si, eb=eb, ib=ib, se=se, sj=sj):
                    for cp in dma_pair(si, eb, ib, se, sj):
                        cp.wait()
                    @pl.when(si == 0)
                    def _():
                        fs_s[0] = ib[pl.ds(0, 16)][0]
                    process_subchunk((s0 + si) * SUB, eb, ib)
                    @pl.when(si + 2 < nsub)
                    def _():
                        for cp in dma_pair(si + 2, eb, ib, se, sj):
                            cp.start()
            return 0

        lax.fori_loop(0, MAXP, pair, 0)

        stage_v[pl.ds(0, 16)] = jnp.full((16,), 1, jnp.int32) * fs_s[0]
        pltpu.sync_copy(stage_v.at[pl.ds(0, 8)],
                        eseg_sh.at[pl.ds(16 * s, 8)])
        stage_v[pl.ds(0, 16)] = jnp.full((16,), 1, jnp.int32) * cur_s[0]
        pltpu.sync_copy(stage_v.at[pl.ds(0, 8)],
                        eseg_sh.at[pl.ds(16 * s + 8, 8)])

        @pl.when(cur_s[0] == fs_s[0])
        def _():
            pltpu.sync_copy(acc_v, emax_sh.at[2 * s])
            pltpu.sync_copy(acc_v, emax_sh.at[2 * s + 1])

        @pl.when(cur_s[0] != fs_s[0])
        def _():
            pltpu.sync_copy(acc_v, emax_sh.at[2 * s + 1])

        plsc.subcore_barrier()

        @pl.when(s == 0)
        def _():
            pltpu.sync_copy(eseg_sh, eseg_v)
            pltpu.sync_copy(emax_sh, emax_v)

            iota16 = lax.iota(jnp.int32, 16)
            es0 = plsc.load_gather(eseg_v, [iota16 * 8])
            es1 = plsc.load_gather(eseg_v, [(iota16 + 16) * 8])
            first = es0[0]

            cur_s[0] = first
            for d in range(DV):
                acc_v[pl.ds(16 * d, 16)] = emax_v[0, pl.ds(16 * d, 16)]

            def cflush():
                cur = cur_s[0]
                @pl.when(cur == first)
                def _():
                    pltpu.sync_copy(acc_v, crout.at[c, 0])
                @pl.when(cur != first)
                def _():
                    pltpu.sync_copy(acc_v, maxout_sh.at[cur])

            for e in range(1, 2 * NS):
                sid = es0[e] if e < 16 else es1[e - 16]
                @pl.when(sid == cur_s[0])
                def _(sid=sid, e=e):
                    for d in range(DV):
                        acc_v[pl.ds(16 * d, 16)] = jnp.maximum(
                            acc_v[pl.ds(16 * d, 16)],
                            emax_v[e, pl.ds(16 * d, 16)])
                @pl.when(sid != cur_s[0])
                def _(sid=sid, e=e):
                    cflush()
                    cur_s[0] = sid
                    for d in range(DV):
                        acc_v[pl.ds(16 * d, 16)] = emax_v[e, pl.ds(16 * d, 16)]

            @pl.when(cur_s[0] == first)
            def _():
                pltpu.sync_copy(acc_v, crout.at[c, 0])
            pltpu.sync_copy(acc_v, crout.at[c, 1])

            stage_v[pl.ds(0, 16)] = jnp.full((16,), 1, jnp.int32) * first
            pltpu.sync_copy(stage_v.at[pl.ds(0, 8)],
                            csout.at[pl.ds(c * 16, 8)])
            stage_v[pl.ds(0, 16)] = jnp.full((16,), 1, jnp.int32) * cur_s[0]
            pltpu.sync_copy(stage_v.at[pl.ds(0, 8)],
                            csout.at[pl.ds(c * 16 + 8, 8)])

            pltpu.sync_copy(head_sh, hout.at[c])

        plsc.subcore_barrier()

        pltpu.sync_copy(maxout_sh.at[pl.ds(GB * s, GB)],
                        mout.at[c, pl.ds(GB * s, GB)])

    return sck(all_node_emb, graph_batch)


def _sc_gather(all_node_emb, headout, B):
    N, D = all_node_emb.shape
    NS = 16
    GB = B // NS

    mesh = plsc.VectorSubcoreMesh(
        core_axis_name="c", subcore_axis_name="s", num_cores=1)

    out_type = (
        jax.ShapeDtypeStruct((B, D), jnp.float32),
        jax.ShapeDtypeStruct((B, D), jnp.float32),
    )

    scratch_types = [
        pltpu.VMEM((8 * B,), jnp.int32),
        pltpu.VMEM((8 * B,), jnp.int32),
        pltpu.VMEM((B,), jnp.int32),
        pltpu.VMEM((B,), jnp.int32),
        pltpu.VMEM((B,), jnp.int32),
        pltpu.VMEM((GB,), jnp.int32),
        pltpu.VMEM((GB,), jnp.int32),
        pltpu.VMEM((GB, D), jnp.float32),
        pltpu.VMEM((GB, D), jnp.float32),
        pltpu.SemaphoreType.DMA,
        pltpu.SemaphoreType.DMA,
        pltpu.VMEM_SHARED((B,), jnp.int32),
        pltpu.VMEM_SHARED((B,), jnp.int32),
    ]

    @functools.partial(
        pl.kernel, out_type=out_type, mesh=mesh, scratch_types=scratch_types,
        compiler_params=pltpu.CompilerParams(needs_layout_passes=False))
    def gk(emb_hbm, hin, hout, tout,
           h0_v, h1_v, hc_v, hg_v, tg_v, idx_v, idx2_v, row_v, row2_v,
           sem, sem2, hg_sh, tg_sh):
        t = lax.axis_index("s")

        @pl.when(t == 0)
        def _():
            pltpu.sync_copy(hin.at[0], h0_v)
            pltpu.sync_copy(hin.at[1], h1_v)

            def mn(j, _):
                h0_v[pl.ds(16 * j, 16)] = jnp.minimum(
                    h0_v[pl.ds(16 * j, 16)], h1_v[pl.ds(16 * j, 16)])
                return 0
            lax.fori_loop(0, 8 * B // 16, mn, 0)

            iota16 = lax.iota(jnp.int32, 16)

            def bf(i, carry):
                ci = B // 16 - 1 - i
                idx = (ci * 16 + iota16) * 8
                v = plsc.load_gather(h0_v, [idx])
                r = lax.rev(-v, dimensions=(0,))
                cc = jnp.maximum(plsc.cummax(r), carry)
                hc_v[pl.ds(ci * 16, 16)] = -lax.rev(cc, dimensions=(0,))
                return cc[15]
            lax.fori_loop(0, B // 16, bf, jnp.int32(-N))

            def gi(i, _):
                v = hc_v[pl.ds(16 * i, 16)]
                hg_v[pl.ds(16 * i, 16)] = jnp.minimum(v, N - 1)
                tg_v[pl.ds(16 * i, 16)] = jnp.minimum(v + 1, N - 1)
                return 0
            lax.fori_loop(0, B // 16, gi, 0)
            pltpu.sync_copy(hg_v, hg_sh)
            pltpu.sync_copy(tg_v, tg_sh)

        plsc.subcore_barrier()

        pltpu.sync_copy(hg_sh.at[pl.ds(GB * t, GB)], idx_v)
        pltpu.sync_copy(tg_sh.at[pl.ds(GB * t, GB)], idx2_v)
        cph = pltpu.async_copy(emb_hbm.at[idx_v], row_v, sem)
        cpt = pltpu.async_copy(emb_hbm.at[idx2_v], row2_v, sem2)
        cph.wait()
        pltpu.sync_copy(row_v, hout.at[pl.ds(GB * t, GB)])
        cpt.wait()
        pltpu.sync_copy(row2_v, tout.at[pl.ds(GB * t, GB)])

    return gk(all_node_emb, headout)


def _tc_proj(hrows, trows, maxout2, cseg_f, crow_f, W, b2):
    B, D = hrows.shape

    def body(h_ref, t_ref, m_ref, s_ref, r_ref, w_ref, b_ref, o_ref):
        maxp = jnp.maximum(m_ref[0], m_ref[1])
        segs = s_ref[...]
        rows = r_ref[...]
        iota_b = lax.broadcasted_iota(jnp.int32, (B, 1), 0)
        for j in range(4):
            mask = iota_b == segs[j:j + 1, 0:1]
            maxp = jnp.where(mask, jnp.maximum(maxp, rows[j:j + 1, :]), maxp)

        w = w_ref[...]
        dn = (((1,), (1,)), ((), ()))
        acc = lax.dot_general(h_ref[...], w[:, 0:D], dn,
                              preferred_element_type=jnp.float32)
        acc += lax.dot_general(t_ref[...], w[:, D:2 * D], dn,
                               preferred_element_type=jnp.float32)
        acc += lax.dot_general(maxp, w[:, 2 * D:3 * D], dn,
                               preferred_element_type=jnp.float32)
        o_ref[...] = acc + b_ref[...]

    return pl.pallas_call(
        body,
        out_shape=jax.ShapeDtypeStruct((B, D), jnp.float32),
    )(hrows, trows, maxout2, cseg_f, crow_f, W, b2)


def kernel(all_node_emb, supernode_edge_index, supernode_idx, graph_batch, W, b):
    B = supernode_idx.shape[0]
    D = all_node_emb.shape[1]
    maxout2, headout, cseg, crow = _sc_scan(all_node_emb, graph_batch, B)
    hrows, trows = _sc_gather(all_node_emb, headout, B)
    return _tc_proj(hrows, trows, maxout2,
                    cseg.reshape(4, 8), crow.reshape(4, D),
                    W, b.reshape(1, D))
